# Initial kernel scaffold; baseline (speedup 1.0000x reference)
import jax
import jax.numpy as jnp
from jax.experimental import pallas as pl


def kernel(q, k, v, wq, bq, wk, bk, wv, bv, gamma, beta):
    raise NotImplementedError("write your pallas kernel here")


# speedup vs baseline: 1.2717x; 1.2717x over previous
import math
import functools

import jax
import jax.numpy as jnp
from jax import lax
from jax.experimental import pallas as pl
from jax.experimental.pallas import tpu as pltpu


def _mha_fused_kernel(q_ref, k_ref, v_ref, vres_ref,
                      wqT_ref, bq_ref, wkT_ref, bk_ref, wvT_ref, bv_ref,
                      gamma_ref, beta_ref,
                      out_ref, attn_ref,
                      *, n_head, d_k, inv_scale, eps):
    H, dk = n_head, d_k
    q = q_ref[0]
    k = k_ref[0]
    v = v_ref[0]

    qp = jnp.dot(q, wqT_ref[...], preferred_element_type=jnp.float32) + bq_ref[0]
    kp = jnp.dot(k, wkT_ref[...], preferred_element_type=jnp.float32) + bk_ref[0]
    vp = jnp.dot(v, wvT_ref[...], preferred_element_type=jnp.float32) + bv_ref[0]

    qpb = qp.astype(jnp.bfloat16)
    kpb = kp.astype(jnp.bfloat16)
    vpb = vp.astype(jnp.bfloat16)

    qh = jnp.stack([qpb[:, h * dk:(h + 1) * dk] for h in range(H)], axis=0)
    kh = jnp.stack([kpb[:, h * dk:(h + 1) * dk] for h in range(H)], axis=0)
    vh = jnp.stack([vpb[:, h * dk:(h + 1) * dk] for h in range(H)], axis=0)

    s = jnp.einsum('hqd,hkd->hqk', qh, kh,
                   preferred_element_type=jnp.float32) * inv_scale

    s = s - jnp.max(s, axis=-1, keepdims=True)
    e = jnp.exp(s)
    denom = jnp.sum(e, axis=-1, keepdims=True)
    attn = e * pl.reciprocal(denom)
    attn_ref[0] = attn

    ctx_h = jnp.einsum('hqk,hkd->hqd', attn.astype(jnp.bfloat16), vh,
                       preferred_element_type=jnp.float32)
    ctx = jnp.concatenate([ctx_h[h] for h in range(H)], axis=-1)

    res = ctx + vres_ref[0]
    mean = jnp.mean(res, axis=-1, keepdims=True)
    var = jnp.mean((res - mean) ** 2, axis=-1, keepdims=True)
    normed = (res - mean) * lax.rsqrt(var + eps)
    out_ref[0] = normed * gamma_ref[0] + beta_ref[0]


def kernel(q, k, v, wq, bq, wk, bk, wv, bv, gamma, beta):
    B, S, D = q.shape
    n_head = 8
    d_k = D // n_head
    inv_scale = 1.0 / math.sqrt(d_k)

    qb = q.astype(jnp.bfloat16)
    kb = k.astype(jnp.bfloat16)
    vb = v.astype(jnp.bfloat16)
    wqT = wq.T.astype(jnp.bfloat16)
    wkT = wk.T.astype(jnp.bfloat16)
    wvT = wv.T.astype(jnp.bfloat16)
    bq2 = bq.reshape(1, D)
    bk2 = bk.reshape(1, D)
    bv2 = bv.reshape(1, D)
    g2 = gamma.reshape(1, D)
    b2 = beta.reshape(1, D)

    body = functools.partial(_mha_fused_kernel, n_head=n_head, d_k=d_k,
                             inv_scale=inv_scale, eps=1e-6)

    seq_spec = pl.BlockSpec((1, S, D), lambda b: (b, 0, 0))
    w_spec = pl.BlockSpec((D, D), lambda b: (0, 0))
    vec_spec = pl.BlockSpec((1, D), lambda b: (0, 0))

    out, attn = pl.pallas_call(
        body,
        out_shape=(
            jax.ShapeDtypeStruct((B, S, D), jnp.float32),
            jax.ShapeDtypeStruct((B, n_head, S, S), jnp.float32),
        ),
        grid=(B,),
        in_specs=[
            seq_spec,
            seq_spec,
            seq_spec,
            seq_spec,
            w_spec, vec_spec,
            w_spec, vec_spec,
            w_spec, vec_spec,
            vec_spec, vec_spec,
        ],
        out_specs=[
            pl.BlockSpec((1, S, D), lambda b: (b, 0, 0)),
            pl.BlockSpec((1, n_head, S, S), lambda b: (b, 0, 0, 0)),
        ],
        compiler_params=pltpu.CompilerParams(
            dimension_semantics=("parallel",),
            vmem_limit_bytes=100 * 1024 * 1024,
        ),
    )(qb, kb, vb, v, wqT, bq2, wkT, bk2, wvT, bv2, g2, b2)
    return out, attn


# speedup vs baseline: 1.5998x; 1.2581x over previous
import math
import functools

import jax
import jax.numpy as jnp
from jax import lax
from jax.experimental import pallas as pl
from jax.experimental.pallas import tpu as pltpu


def _mha_fused_kernel(q_ref, k_ref, v_ref,
                      wqT_ref, bq_ref, wkT_ref, bk_ref, wvT_ref, bv_ref,
                      gamma_ref, beta_ref,
                      out_ref, attn_ref,
                      *, n_head, d_k, inv_scale, eps):
    H, dk = n_head, d_k
    q = q_ref[0].astype(jnp.bfloat16)
    k = k_ref[0].astype(jnp.bfloat16)
    v = v_ref[0].astype(jnp.bfloat16)

    qp = jnp.dot(q, wqT_ref[...], preferred_element_type=jnp.float32) + bq_ref[0]
    kp = jnp.dot(k, wkT_ref[...], preferred_element_type=jnp.float32) + bk_ref[0]
    vp = jnp.dot(v, wvT_ref[...], preferred_element_type=jnp.float32) + bv_ref[0]

    qpb = qp.astype(jnp.bfloat16)
    kpb = kp.astype(jnp.bfloat16)
    vpb = vp.astype(jnp.bfloat16)

    qh = jnp.stack([qpb[:, h * dk:(h + 1) * dk] for h in range(H)], axis=0)
    kh = jnp.stack([kpb[:, h * dk:(h + 1) * dk] for h in range(H)], axis=0)
    vh = jnp.stack([vpb[:, h * dk:(h + 1) * dk] for h in range(H)], axis=0)

    s = jnp.einsum('hqd,hkd->hqk', qh, kh,
                   preferred_element_type=jnp.float32) * inv_scale

    s = s - jnp.max(s, axis=-1, keepdims=True)
    e = jnp.exp(s)
    denom = jnp.sum(e, axis=-1, keepdims=True)
    attn = e * pl.reciprocal(denom)
    attn_ref[0] = attn

    ctx_h = jnp.einsum('hqk,hkd->hqd', attn.astype(jnp.bfloat16), vh,
                       preferred_element_type=jnp.float32)
    ctx = jnp.concatenate([ctx_h[h] for h in range(H)], axis=-1)

    res = ctx + v_ref[0]
    mean = jnp.mean(res, axis=-1, keepdims=True)
    var = jnp.mean((res - mean) ** 2, axis=-1, keepdims=True)
    normed = (res - mean) * lax.rsqrt(var + eps)
    out_ref[0] = normed * gamma_ref[0] + beta_ref[0]


def kernel(q, k, v, wq, bq, wk, bk, wv, bv, gamma, beta):
    B, S, D = q.shape
    n_head = 8
    d_k = D // n_head
    inv_scale = 1.0 / math.sqrt(d_k)

    wqT = wq.T.astype(jnp.bfloat16)
    wkT = wk.T.astype(jnp.bfloat16)
    wvT = wv.T.astype(jnp.bfloat16)
    bq2 = bq.reshape(1, D)
    bk2 = bk.reshape(1, D)
    bv2 = bv.reshape(1, D)
    g2 = gamma.reshape(1, D)
    b2 = beta.reshape(1, D)

    body = functools.partial(_mha_fused_kernel, n_head=n_head, d_k=d_k,
                             inv_scale=inv_scale, eps=1e-6)

    seq_spec = pl.BlockSpec((1, S, D), lambda b: (b, 0, 0))
    w_spec = pl.BlockSpec((D, D), lambda b: (0, 0))
    vec_spec = pl.BlockSpec((1, D), lambda b: (0, 0))

    out, attn = pl.pallas_call(
        body,
        out_shape=(
            jax.ShapeDtypeStruct((B, S, D), jnp.float32),
            jax.ShapeDtypeStruct((B, n_head, S, S), jnp.float32),
        ),
        grid=(B,),
        in_specs=[
            seq_spec,
            seq_spec,
            seq_spec,
            w_spec, vec_spec,
            w_spec, vec_spec,
            w_spec, vec_spec,
            vec_spec, vec_spec,
        ],
        out_specs=[
            pl.BlockSpec((1, S, D), lambda b: (b, 0, 0)),
            pl.BlockSpec((1, n_head, S, S), lambda b: (b, 0, 0, 0)),
        ],
        compiler_params=pltpu.CompilerParams(
            dimension_semantics=("parallel",),
            vmem_limit_bytes=100 * 1024 * 1024,
        ),
    )(q, k, v, wqT, bq2, wkT, bk2, wvT, bv2, g2, b2)
    return out, attn
